# Initial kernel scaffold; baseline (speedup 1.0000x reference)
import jax
import jax.numpy as jnp
from jax.experimental import pallas as pl


def kernel(t, data, edges, pos, edge_attr, W1, b1, W2, b2, W3, b3):
    raise NotImplementedError("write your pallas kernel here")


# speedup vs baseline: 10.3044x; 10.3044x over previous
import functools

import jax
import jax.numpy as jnp
from jax import lax
from jax.experimental import pallas as pl
from jax.experimental.pallas import tpu as pltpu
from jax.experimental.pallas import tpu_sc as plsc

N = 10000
NP = 10240
E = 320000
NC, NS = 2, 16
NW = NC * NS
EPW = E // NW
B = 80
NB = EPW // B
RPT = NP // NS


def _edge_scatter(C):
  mesh = plsc.VectorSubcoreMesh(core_axis_name="c", subcore_axis_name="s")

  @functools.partial(
      pl.kernel,
      out_type=jax.ShapeDtypeStruct((NC, NP, C), jnp.float32),
      mesh=mesh,
      scratch_types=[
          pltpu.VMEM((EPW,), jnp.int32),
          pltpu.VMEM((EPW,), jnp.float32),
          pltpu.VMEM((B,), jnp.int32),
          pltpu.VMEM((B, C), jnp.float32),
          pltpu.VMEM_SHARED((NP, C), jnp.float32),
          pltpu.SemaphoreType.DMA,
      ],
      compiler_params=pltpu.CompilerParams(use_tc_tiling_on_sc=False),
  )
  def k(y_hbm, row_hbm, col_hbm, ew_hbm, z_hbm, out_hbm,
        row_v, ew_v, col_b, msg_v, acc, sem):
    cid = lax.axis_index("c")
    sid = lax.axis_index("s")
    wid = sid * NC + cid
    ebase = pl.multiple_of(wid * EPW, 8)
    pltpu.sync_copy(row_hbm.at[pl.ds(ebase, EPW)], row_v)
    pltpu.sync_copy(ew_hbm.at[pl.ds(ebase, EPW)], ew_v)
    rbase = pl.multiple_of(sid * RPT, 8)
    pltpu.sync_copy(z_hbm.at[pl.ds(rbase, RPT)], acc.at[pl.ds(rbase, RPT)])
    plsc.subcore_barrier()

    def batch_body(b, carry):
      off = pl.multiple_of(b * B, 8)
      pltpu.async_copy(y_hbm.at[row_v.at[pl.ds(off, B)]], msg_v, sem).wait()
      pltpu.sync_copy(col_hbm.at[pl.ds(ebase + off, B)], col_b)

      def scale(g, c2):
        ew16 = ew_v[pl.ds(off + g * 16, 16)]
        for j in range(16):
          e = g * 16 + j
          s = ew16.at[jnp.full((16,), j, jnp.int32)].get(
              mode="promise_in_bounds")
          for cc in range(C // 16):
            msg_v[e, pl.ds(cc * 16, 16)] = msg_v[e, pl.ds(cc * 16, 16)] * s
        return c2

      lax.fori_loop(0, B // 16, scale, 0)
      pltpu.sync_copy(msg_v, acc.at[col_b], add=True)
      return carry

    lax.fori_loop(0, NB, batch_body, 0)
    plsc.subcore_barrier()
    pltpu.sync_copy(acc.at[pl.ds(rbase, RPT)],
                    out_hbm.at[cid, pl.ds(rbase, RPT)])

  return k


_scatter16 = _edge_scatter(16)
_scatter64 = _edge_scatter(64)
_scatter32 = _edge_scatter(32)


R_BLK = 400
GRID = N // R_BLK


def _row_spec(c):
  return pl.BlockSpec((R_BLK, c), lambda i: (i, 0))


def _full_spec(r, c):
  return pl.BlockSpec((r, c), lambda i: (0, 0))


def _part_spec(c):
  return pl.BlockSpec((2, R_BLK, c), lambda i: (0, i, 0))


def _tc1(data, w1r, tw, deg2):
  def body(d_ref, w_ref, tw_ref, dg_ref, y0_ref, sl1_ref, dinv_ref):
    xw = jnp.dot(d_ref[...], w_ref[...],
                 preferred_element_type=jnp.float32) + tw_ref[...]
    deg = dg_ref[0, :, 0:1] + dg_ref[1, :, 0:1] + 1.0
    dinv = jnp.where(deg > 0, lax.rsqrt(deg), 0.0)
    y0_ref[...] = dinv * xw
    sl1_ref[...] = (dinv * dinv) * xw
    dinv_ref[...] = dinv

  return pl.pallas_call(
      body,
      grid=(GRID,),
      in_specs=[_row_spec(128), _full_spec(128, 64), _full_spec(1, 64),
                _part_spec(16)],
      out_specs=[_row_spec(64), _row_spec(64), _row_spec(1)],
      out_shape=[
          jax.ShapeDtypeStruct((N, 64), jnp.float32),
          jax.ShapeDtypeStruct((N, 64), jnp.float32),
          jax.ShapeDtypeStruct((N, 1), jnp.float32),
      ],
  )(data, w1r, tw, deg2)


def _tc2(s1, sl1, dinv, b1, w2):
  def body(s_ref, sl_ref, dv_ref, b_ref, w_ref, y_ref, sl2_ref):
    dinv = dv_ref[...]
    h = jnp.tanh(dinv * (s_ref[0] + s_ref[1]) + sl_ref[...] + b_ref[...])
    xw = jnp.dot(h, w_ref[...], preferred_element_type=jnp.float32)
    y_ref[...] = dinv * xw
    sl2_ref[...] = (dinv * dinv) * xw

  return pl.pallas_call(
      body,
      grid=(GRID,),
      in_specs=[_part_spec(64), _row_spec(64), _row_spec(1),
                _full_spec(1, 64), _full_spec(64, 32)],
      out_specs=[_row_spec(32), _row_spec(32)],
      out_shape=[
          jax.ShapeDtypeStruct((N, 32), jnp.float32),
          jax.ShapeDtypeStruct((N, 32), jnp.float32),
      ],
  )(s1, sl1, dinv, b1, w2)


def _tc3(s2, sl2, dinv, b2):
  def body(s_ref, sl_ref, dv_ref, b_ref, y_ref, sl3_ref):
    dinv = dv_ref[...]
    h = jnp.tanh(dinv * (s_ref[0] + s_ref[1]) + sl_ref[...] + b_ref[...])
    y_ref[...] = dinv * h
    sl3_ref[...] = (dinv * dinv) * h

  return pl.pallas_call(
      body,
      grid=(GRID,),
      in_specs=[_part_spec(32), _row_spec(32), _row_spec(1),
                _full_spec(1, 32)],
      out_specs=[_row_spec(32), _row_spec(32)],
      out_shape=[
          jax.ShapeDtypeStruct((N, 32), jnp.float32),
          jax.ShapeDtypeStruct((N, 32), jnp.float32),
      ],
  )(s2, sl2, dinv, b2)


def _tc4(s3, sl3, dinv, w3, b3):
  def body(s_ref, sl_ref, dv_ref, w_ref, b_ref, o_ref):
    agg = dv_ref[...] * (s_ref[0] + s_ref[1]) + sl_ref[...]
    o_ref[...] = jnp.dot(agg, w_ref[...],
                         preferred_element_type=jnp.float32) + b_ref[...]

  return pl.pallas_call(
      body,
      grid=(GRID,),
      in_specs=[_part_spec(32), _row_spec(32), _row_spec(1),
                _full_spec(32, 128), _full_spec(1, 128)],
      out_specs=_row_spec(128),
      out_shape=jax.ShapeDtypeStruct((N, 128), jnp.float32),
  )(s3, sl3, dinv, w3, b3)


def kernel(t, data, edges, pos, edge_attr, W1, b1, W2, b2, W3, b3):
  del pos
  edges = edges.astype(jnp.int32)
  row, col = edges[0], edges[1]
  ew = edge_attr.astype(jnp.float32)
  data = data.astype(jnp.float32)

  ones16 = jnp.ones((N, 16), jnp.float32)
  z16 = jnp.zeros((NP, 16), jnp.float32)
  z64 = jnp.zeros((NP, 64), jnp.float32)
  z32 = jnp.zeros((NP, 32), jnp.float32)
  tw = (t * W1[0])[None, :]
  w1r = W1[1:]

  deg2 = _scatter16(ones16, row, col, ew, z16)[:, :N]
  y0, sl1, dinv = _tc1(data, w1r, tw, deg2)
  s1 = _scatter64(y0, row, col, ew, z64)[:, :N]
  y1, sl2 = _tc2(s1, sl1, dinv, b1[None, :], W2)
  s2 = _scatter32(y1, row, col, ew, z32)[:, :N]
  y2, sl3 = _tc3(s2, sl2, dinv, b2[None, :])
  s3 = _scatter32(y2, row, col, ew, z32)[:, :N]
  return _tc4(s3, sl3, dinv, W3, b3[None, :])


# speedup vs baseline: 13.8472x; 1.3438x over previous
import functools

import jax
import jax.numpy as jnp
from jax import lax
from jax.experimental import pallas as pl
from jax.experimental.pallas import tpu as pltpu
from jax.experimental.pallas import tpu_sc as plsc

N = 10000
NP = 10240
E = 320000
NC, NS = 2, 16
NW = NC * NS
B = 128
NB = 80
EPW = B * NB
EPAD = EPW * NW
NSLOT = 4
RPT = NP // NS


def _edge_scatter(C, deg_mode=False):
  mesh = plsc.VectorSubcoreMesh(core_axis_name="c", subcore_axis_name="s")

  scratch = [
      pltpu.VMEM((EPW,), jnp.int32),
      pltpu.VMEM((EPW,), jnp.float32),
      pltpu.VMEM_SHARED((NP, C), jnp.float32),
  ]
  for _ in range(NSLOT):
    scratch.append(pltpu.VMEM((B,), jnp.int32))
  for _ in range(NSLOT):
    scratch.append(pltpu.VMEM((B, C), jnp.float32))
  scratch += [pltpu.SemaphoreType.DMA] * (2 * NSLOT)

  @functools.partial(
      pl.kernel,
      out_type=jax.ShapeDtypeStruct((NC, NP, C), jnp.float32),
      mesh=mesh,
      scratch_types=scratch,
      compiler_params=pltpu.CompilerParams(use_tc_tiling_on_sc=False),
  )
  def k(y_hbm, row_hbm, col_hbm, ew_hbm, z_hbm, out_hbm, row_v, ew_v, acc,
        *bufs):
    colb = bufs[0:NSLOT]
    msg = bufs[NSLOT:2 * NSLOT]
    gsem = bufs[2 * NSLOT:3 * NSLOT]
    ssem = bufs[3 * NSLOT:4 * NSLOT]
    cid = lax.axis_index("c")
    sid = lax.axis_index("s")
    wid = sid * NC + cid
    ebase = pl.multiple_of(wid * EPW, 8)
    pltpu.sync_copy(row_hbm.at[pl.ds(ebase, EPW)], row_v)
    pltpu.sync_copy(ew_hbm.at[pl.ds(ebase, EPW)], ew_v)
    rbase = pl.multiple_of(sid * RPT, 8)
    pltpu.sync_copy(z_hbm.at[pl.ds(rbase, RPT)], acc.at[pl.ds(rbase, RPT)])
    plsc.subcore_barrier()

    def gather_start(b, j):
      off = pl.multiple_of(b * B, 8)
      pltpu.async_copy(col_hbm.at[pl.ds(ebase + off, B)], colb[j], gsem[j])
      if not deg_mode:
        pltpu.async_copy(y_hbm.at[row_v.at[pl.ds(off, B)]], msg[j], gsem[j])

    def gather_wait(j):
      pltpu.make_async_copy(col_hbm.at[pl.ds(0, B)], colb[j], gsem[j]).wait()
      if not deg_mode:
        pltpu.make_async_copy(y_hbm.at[row_v.at[pl.ds(0, B)]], msg[j],
                              gsem[j]).wait()

    def scatter_wait(j):
      pltpu.make_async_copy(msg[j], acc.at[colb[j]], ssem[j]).wait()

    def scale(b, j):
      off = pl.multiple_of(b * B, 8)

      def grp(g, c2):
        ew16 = ew_v[pl.ds(off + g * 16, 16)]
        for jj in range(16):
          e = g * 16 + jj
          s = ew16.at[jnp.full((16,), jj, jnp.int32)].get(
              mode="promise_in_bounds")
          if deg_mode:
            msg[j][e, pl.ds(0, 16)] = s
          else:
            for cc in range(C // 16):
              msg[j][e, pl.ds(cc * 16, 16)] = msg[j][e, pl.ds(cc * 16, 16)] * s
        return c2

      lax.fori_loop(0, B // 16, grp, 0)

    gather_start(0, 0)
    gather_start(1, 1)

    def outer(i, carry):
      for jj in range(NSLOT):
        b = i * NSLOT + jj
        gather_wait(jj)
        scale(b, jj)
        pltpu.async_copy(msg[jj], acc.at[colb[jj]], ssem[jj], add=True)
        j2 = (jj + 2) % NSLOT

        @pl.when(b >= 2)
        def _():
          scatter_wait(j2)

        @pl.when(b + 2 < NB)
        def _():
          gather_start(b + 2, j2)

      return carry

    lax.fori_loop(0, NB // NSLOT, outer, 0)
    scatter_wait((NB - 2) % NSLOT)
    scatter_wait((NB - 1) % NSLOT)
    plsc.subcore_barrier()
    pltpu.sync_copy(acc.at[pl.ds(rbase, RPT)],
                    out_hbm.at[cid, pl.ds(rbase, RPT)])

  return k


_scatter_deg = _edge_scatter(16)
_scatter64 = _edge_scatter(64)
_scatter32 = _edge_scatter(32)


R_BLK = 400
GRID = N // R_BLK


def _row_spec(c):
  return pl.BlockSpec((R_BLK, c), lambda i: (i, 0))


def _full_spec(r, c):
  return pl.BlockSpec((r, c), lambda i: (0, 0))


def _part_spec(c):
  return pl.BlockSpec((2, R_BLK, c), lambda i: (0, i, 0))


def _tc1(data, w1r, tw, deg2):
  def body(d_ref, w_ref, tw_ref, dg_ref, y0_ref, sl1_ref, dinv_ref):
    xw = jnp.dot(d_ref[...], w_ref[...],
                 preferred_element_type=jnp.float32) + tw_ref[...]
    deg = dg_ref[0, :, 0:1] + dg_ref[1, :, 0:1] + 1.0
    dinv = jnp.where(deg > 0, lax.rsqrt(deg), 0.0)
    y0_ref[...] = dinv * xw
    sl1_ref[...] = (dinv * dinv) * xw
    dinv_ref[...] = dinv

  return pl.pallas_call(
      body,
      grid=(GRID,),
      in_specs=[_row_spec(128), _full_spec(128, 64), _full_spec(1, 64),
                _part_spec(16)],
      out_specs=[_row_spec(64), _row_spec(64), _row_spec(1)],
      out_shape=[
          jax.ShapeDtypeStruct((N, 64), jnp.float32),
          jax.ShapeDtypeStruct((N, 64), jnp.float32),
          jax.ShapeDtypeStruct((N, 1), jnp.float32),
      ],
  )(data, w1r, tw, deg2)


def _tc2(s1, sl1, dinv, b1, w2):
  def body(s_ref, sl_ref, dv_ref, b_ref, w_ref, y_ref, sl2_ref):
    dinv = dv_ref[...]
    h = jnp.tanh(dinv * (s_ref[0] + s_ref[1]) + sl_ref[...] + b_ref[...])
    xw = jnp.dot(h, w_ref[...], preferred_element_type=jnp.float32)
    y_ref[...] = dinv * xw
    sl2_ref[...] = (dinv * dinv) * xw

  return pl.pallas_call(
      body,
      grid=(GRID,),
      in_specs=[_part_spec(64), _row_spec(64), _row_spec(1),
                _full_spec(1, 64), _full_spec(64, 32)],
      out_specs=[_row_spec(32), _row_spec(32)],
      out_shape=[
          jax.ShapeDtypeStruct((N, 32), jnp.float32),
          jax.ShapeDtypeStruct((N, 32), jnp.float32),
      ],
  )(s1, sl1, dinv, b1, w2)


def _tc3(s2, sl2, dinv, b2):
  def body(s_ref, sl_ref, dv_ref, b_ref, y_ref, sl3_ref):
    dinv = dv_ref[...]
    h = jnp.tanh(dinv * (s_ref[0] + s_ref[1]) + sl_ref[...] + b_ref[...])
    y_ref[...] = dinv * h
    sl3_ref[...] = (dinv * dinv) * h

  return pl.pallas_call(
      body,
      grid=(GRID,),
      in_specs=[_part_spec(32), _row_spec(32), _row_spec(1),
                _full_spec(1, 32)],
      out_specs=[_row_spec(32), _row_spec(32)],
      out_shape=[
          jax.ShapeDtypeStruct((N, 32), jnp.float32),
          jax.ShapeDtypeStruct((N, 32), jnp.float32),
      ],
  )(s2, sl2, dinv, b2)


def _tc4(s3, sl3, dinv, w3, b3):
  def body(s_ref, sl_ref, dv_ref, w_ref, b_ref, o_ref):
    agg = dv_ref[...] * (s_ref[0] + s_ref[1]) + sl_ref[...]
    o_ref[...] = jnp.dot(agg, w_ref[...],
                         preferred_element_type=jnp.float32) + b_ref[...]

  return pl.pallas_call(
      body,
      grid=(GRID,),
      in_specs=[_part_spec(32), _row_spec(32), _row_spec(1),
                _full_spec(32, 128), _full_spec(1, 128)],
      out_specs=_row_spec(128),
      out_shape=jax.ShapeDtypeStruct((N, 128), jnp.float32),
  )(s3, sl3, dinv, w3, b3)


def kernel(t, data, edges, pos, edge_attr, W1, b1, W2, b2, W3, b3):
  del pos
  edges = edges.astype(jnp.int32)
  pad = jnp.zeros((2, EPAD - E), jnp.int32)
  edges = jnp.concatenate([edges, pad], axis=1)
  row, col = edges[0], edges[1]
  ew = jnp.concatenate(
      [edge_attr.astype(jnp.float32), jnp.zeros((EPAD - E,), jnp.float32)])
  data = data.astype(jnp.float32)

  ones16 = jnp.ones((N, 16), jnp.float32)
  z16 = jnp.zeros((NP, 16), jnp.float32)
  z64 = jnp.zeros((NP, 64), jnp.float32)
  z32 = jnp.zeros((NP, 32), jnp.float32)
  tw = (t * W1[0])[None, :]
  w1r = W1[1:]

  deg2 = _scatter_deg(ones16, row, col, ew, z16)[:, :N]
  y0, sl1, dinv = _tc1(data, w1r, tw, deg2)
  s1 = _scatter64(y0, row, col, ew, z64)[:, :N]
  y1, sl2 = _tc2(s1, sl1, dinv, b1[None, :], W2)
  s2 = _scatter32(y1, row, col, ew, z32)[:, :N]
  y2, sl3 = _tc3(s2, sl2, dinv, b2[None, :])
  s3 = _scatter32(y2, row, col, ew, z32)[:, :N]
  return _tc4(s3, sl3, dinv, W3, b3[None, :])
